# Initial kernel scaffold; baseline (speedup 1.0000x reference)
import jax
import jax.numpy as jnp
from jax.experimental import pallas as pl


def kernel(x, edge_index, batch, proj_W0, proj_b0, proj_W1, proj_b1, rel_W0, rel_b0, root_W0, ln_g0, ln_b0, rel_W1, rel_b1, root_W1, ln_g1, ln_b1):
    raise NotImplementedError("write your pallas kernel here")


# speedup vs baseline: 4.8637x; 4.8637x over previous
import functools

import jax
import jax.numpy as jnp
from jax import lax
from jax.experimental import pallas as pl
from jax.experimental.pallas import tpu as pltpu
from jax.experimental.pallas import tpu_sc as plsc

N = 10000
E = 320000
D = 128
G = 64

NC, NS = 2, 16
NW = NC * NS
CH = 128
EPW = E // NW
NCH = -(-EPW // CH)
EPW_PAD = NCH * CH
ROWS_PT = 632
N_PAD = NS * ROWS_PT

_R = 2000
_NG = N // _R


def _leaky(v):
    return jnp.where(v >= 0, v, 0.01 * v)


def _sc_agg(h, src_p, dst_p, zrows):
    mesh = plsc.VectorSubcoreMesh(
        core_axis_name="c", subcore_axis_name="s",
        num_cores=NC, num_subcores=NS)

    @functools.partial(
        pl.kernel,
        out_type=jax.ShapeDtypeStruct((NC, N_PAD, D), jnp.float32),
        mesh=mesh,
        scratch_types=[
            pltpu.VMEM((NCH, CH), jnp.int32),
            pltpu.VMEM((NCH, CH), jnp.int32),
            pltpu.VMEM((CH, D), jnp.float32),
            pltpu.VMEM_SHARED((N_PAD, D), jnp.float32),
            pltpu.SemaphoreType.DMA,
        ],
    )
    def k(h_hbm, src_hbm, dst_hbm, z_hbm, out_hbm, sidx, didx, rows, agg, sem):
        c = lax.axis_index("c")
        s = lax.axis_index("s")
        wid = s * NC + c
        pltpu.sync_copy(z_hbm, agg.at[pl.ds(s * ROWS_PT, ROWS_PT)])
        pltpu.sync_copy(src_hbm.at[wid], sidx)
        pltpu.sync_copy(dst_hbm.at[wid], didx)
        plsc.subcore_barrier()

        def body(j, carry):
            pltpu.async_copy(h_hbm.at[sidx.at[j]], rows, sem).wait()
            pltpu.sync_copy(rows, agg.at[didx.at[j]], add=True)
            return carry

        lax.fori_loop(0, NCH, body, 0)
        plsc.subcore_barrier()
        pltpu.sync_copy(agg.at[pl.ds(s * ROWS_PT, ROWS_PT)],
                        out_hbm.at[c].at[pl.ds(s * ROWS_PT, ROWS_PT)])

    return k(h, src_p, dst_p, zrows)


def _tc_proj(x, W0, b0, W1, b1):
    def body(x_ref, w0_ref, b0_ref, w1_ref, b1_ref, o_ref):
        h = _leaky(lax.dot_general(x_ref[...], w0_ref[...],
                                   (((1,), (1,)), ((), ())),
                                   preferred_element_type=jnp.float32)
                   + b0_ref[...])
        o_ref[...] = _leaky(lax.dot_general(h, w1_ref[...],
                                            (((1,), (1,)), ((), ())),
                                            preferred_element_type=jnp.float32)
                            + b1_ref[...])

    return pl.pallas_call(
        body,
        grid=(_NG,),
        in_specs=[pl.BlockSpec((_R, D), lambda i: (i, 0)),
                  pl.BlockSpec((D, D), lambda i: (0, 0)),
                  pl.BlockSpec((1, D), lambda i: (0, 0)),
                  pl.BlockSpec((D, D), lambda i: (0, 0)),
                  pl.BlockSpec((1, D), lambda i: (0, 0))],
        out_specs=pl.BlockSpec((_R, D), lambda i: (i, 0)),
        out_shape=jax.ShapeDtypeStruct((N, D), jnp.float32),
    )(x, W0, b0.reshape(1, D), W1, b1.reshape(1, D))


def _tc_layer(parts, h, resid, rW, rb, roW, lg, lb):
    def body(a_ref, b_ref, h_ref, r_ref, rw_ref, rb_ref, row_ref,
             lg_ref, lb_ref, o_ref):
        agg = a_ref[0] + b_ref[0]
        new = (lax.dot_general(agg, rw_ref[...], (((1,), (1,)), ((), ())),
                               preferred_element_type=jnp.float32)
               + rb_ref[...]
               + lax.dot_general(h_ref[...], row_ref[...],
                                 (((1,), (1,)), ((), ())),
                                 preferred_element_type=jnp.float32))
        mu = jnp.mean(new, axis=-1, keepdims=True)
        var = jnp.mean((new - mu) ** 2, axis=-1, keepdims=True)
        new = (new - mu) * lax.rsqrt(var + 1e-5) * lg_ref[...] + lb_ref[...]
        o_ref[...] = _leaky(new) + r_ref[...]

    return pl.pallas_call(
        body,
        grid=(_NG,),
        in_specs=[pl.BlockSpec((1, _R, D), lambda i: (0, i, 0)),
                  pl.BlockSpec((1, _R, D), lambda i: (1, i, 0)),
                  pl.BlockSpec((_R, D), lambda i: (i, 0)),
                  pl.BlockSpec((_R, D), lambda i: (i, 0)),
                  pl.BlockSpec((D, D), lambda i: (0, 0)),
                  pl.BlockSpec((1, D), lambda i: (0, 0)),
                  pl.BlockSpec((D, D), lambda i: (0, 0)),
                  pl.BlockSpec((1, D), lambda i: (0, 0)),
                  pl.BlockSpec((1, D), lambda i: (0, 0))],
        out_specs=pl.BlockSpec((_R, D), lambda i: (i, 0)),
        out_shape=jax.ShapeDtypeStruct((N, D), jnp.float32),
    )(parts, parts, h, resid, rW, rb.reshape(1, D), roW,
      lg.reshape(1, D), lb.reshape(1, D))


def _tc_pool(h, batch2d):
    def body(h_ref, b_ref, o_ref, s_sum, s_cnt):
        i = pl.program_id(0)

        @pl.when(i == 0)
        def _():
            s_sum[...] = jnp.zeros_like(s_sum)
            s_cnt[...] = jnp.zeros_like(s_cnt)

        oh = (b_ref[...] == lax.broadcasted_iota(jnp.int32, (_R, G), 1)
              ).astype(jnp.float32)
        s_sum[...] += lax.dot_general(oh, h_ref[...],
                                      (((0,), (0,)), ((), ())),
                                      preferred_element_type=jnp.float32)
        s_cnt[...] += lax.dot_general(oh, jnp.ones((_R, 1), jnp.float32),
                                      (((0,), (0,)), ((), ())),
                                      preferred_element_type=jnp.float32)

        @pl.when(i == _NG - 1)
        def _():
            o_ref[...] = s_sum[...] / jnp.maximum(s_cnt[...], 1.0)

    return pl.pallas_call(
        body,
        grid=(_NG,),
        in_specs=[pl.BlockSpec((_R, D), lambda i: (i, 0)),
                  pl.BlockSpec((_R, 1), lambda i: (i, 0))],
        out_specs=pl.BlockSpec((G, D), lambda i: (0, 0)),
        out_shape=jax.ShapeDtypeStruct((G, D), jnp.float32),
        scratch_shapes=[pltpu.VMEM((G, D), jnp.float32),
                        pltpu.VMEM((G, 1), jnp.float32)],
    )(h, batch2d)


def kernel(x, edge_index, batch, proj_W0, proj_b0, proj_W1, proj_b1,
           rel_W0, rel_b0, root_W0, ln_g0, ln_b0,
           rel_W1, rel_b1, root_W1, ln_g1, ln_b1):
    x = x.astype(jnp.float32)
    src, dst = edge_index[0], edge_index[1]
    pad = NW * EPW_PAD - E
    src_p = jnp.concatenate([src, jnp.zeros((pad,), jnp.int32)]
                            ).reshape(NW, NCH, CH)
    dst_p = jnp.concatenate([dst, jnp.full((pad,), N, jnp.int32)]
                            ).reshape(NW, NCH, CH)
    zrows = jnp.zeros((ROWS_PT, D), jnp.float32)

    h = _tc_proj(x, proj_W0, proj_b0, proj_W1, proj_b1)
    parts = _sc_agg(h, src_p, dst_p, zrows)
    h1 = _tc_layer(parts, h, x, rel_W0, rel_b0, root_W0, ln_g0, ln_b0)
    parts = _sc_agg(h1, src_p, dst_p, zrows)
    h2 = _tc_layer(parts, h1, h1, rel_W1, rel_b1, root_W1, ln_g1, ln_b1)
    return _tc_pool(h2, batch.reshape(N, 1))
